# baseline (speedup 1.0000x reference)
import functools

import jax
import jax.numpy as jnp
from jax import lax
from jax.experimental import pallas as pl
from jax.experimental.pallas import tpu as pltpu
from jax.experimental.pallas import tpu_sc as plsc

BATCH = 16384
EMB_DIM = 64
NUM_CORES = 2
NUM_SUBCORES = 16
NUM_WORKERS = NUM_CORES * NUM_SUBCORES
ROWS_PER_WORKER = BATCH // NUM_WORKERS
GATHER_CHUNK = 128
NUM_CHUNKS = ROWS_PER_WORKER // GATHER_CHUNK
LANES = 16
VECS_PER_ROW = EMB_DIM // LANES


def _body(feat_hbm, lab_hbm, cent_hbm, out_hbm, idx_v, cent_v, feat_v, acc_v,
          gsem, fsem):
    wid = lax.axis_index("s") * NUM_CORES + lax.axis_index("c")
    base = wid * ROWS_PER_WORKER

    pltpu.sync_copy(lab_hbm.at[wid], idx_v)
    fcopy = pltpu.async_copy(feat_hbm.at[pl.ds(base, ROWS_PER_WORKER)],
                             feat_v, fsem)
    gathers = []
    for k in range(NUM_CHUNKS):
        gathers.append(
            pltpu.async_copy(
                cent_hbm.at[idx_v.at[k]],
                cent_v.at[pl.ds(k * GATHER_CHUNK, GATHER_CHUNK)],
                gsem))
    fcopy.wait()
    for g in gathers:
        g.wait()

    zeros = jnp.zeros((LANES,), jnp.float32)

    def row(i, accs):
        new = []
        for j in range(VECS_PER_ROW):
            f = feat_v[i, pl.ds(j * LANES, LANES)]
            c = cent_v[i, pl.ds(j * LANES, LANES)]
            d = f - c
            new.append(accs[j] + d * d)
        return tuple(new)

    accs = lax.fori_loop(0, ROWS_PER_WORKER, row,
                         (zeros,) * VECS_PER_ROW, unroll=2)
    acc_v[...] = (accs[0] + accs[1]) + (accs[2] + accs[3])
    pltpu.sync_copy(acc_v, out_hbm.at[wid])


@jax.jit
def _center_loss(features, labels, centers):
    labels_tiled = labels.astype(jnp.int32).reshape(
        NUM_WORKERS, NUM_CHUNKS, GATHER_CHUNK)
    mesh = plsc.VectorSubcoreMesh(core_axis_name="c", subcore_axis_name="s")
    partials = pl.kernel(
        _body,
        out_type=jax.ShapeDtypeStruct((NUM_WORKERS, LANES), jnp.float32),
        mesh=mesh,
        scratch_types=[
            pltpu.VMEM((NUM_CHUNKS, GATHER_CHUNK), jnp.int32),
            pltpu.VMEM((ROWS_PER_WORKER, EMB_DIM), jnp.float32),
            pltpu.VMEM((ROWS_PER_WORKER, EMB_DIM), jnp.float32),
            pltpu.VMEM((LANES,), jnp.float32),
            pltpu.SemaphoreType.DMA,
            pltpu.SemaphoreType.DMA,
        ],
        compiler_params=pltpu.CompilerParams(use_tc_tiling_on_sc=False),
    )(features, labels_tiled, centers)
    return jnp.sum(partials) / (2.0 * features.shape[0])


def kernel(features, labels, centers):
    return _center_loss(features, labels, centers)
